# baseline (speedup 1.0000x reference)
import functools

import jax
import jax.numpy as jnp
from jax import lax
from jax.experimental import pallas as pl
from jax.experimental.pallas import tpu as pltpu
from jax.experimental.pallas import tpu_sc as plsc

_N = 10000
_E = 320000
_H = 128
_L = 3

_NC = 2
_NS = 16
_NW = _NC * _NS
_K = 80
_NCHUNK = 125
_NH = _N // 2
_NTRASH = 8
_NACC = _NH + _NTRASH
_ZR = 8
_NZCH = _NH // _ZR


def _sc_agg_half_body(base, pe_hbm, h_hbm, out_hbm,
                      pk_v, src_v, dst_v, rows0, rows1, zbuf,
                      sem0, sem1, acc_sh):
    c = lax.axis_index("c")
    s = lax.axis_index("s")
    wid = s * _NC + c

    pltpu.sync_copy(pe_hbm.at[wid], pk_v)
    lanes = lax.iota(jnp.int32, 16)
    trash = _NH + lax.bitwise_and(lanes, _NTRASH - 1)

    def _unpack(ch, carry):
        for g in range(_K // 16):
            sl = pl.ds(g * 16, 16)
            v = pk_v[ch, sl]
            src_v[ch, sl] = lax.bitwise_and(v, 0xFFFF)
            t = lax.shift_right_logical(v, 16) - base
            bad = (t < 0) | (t >= _NH)
            dst_v[ch, sl] = jnp.where(bad, trash, t)
        return carry
    lax.fori_loop(0, _NCHUNK, _unpack, 0)

    def _zrow(r, carry):
        for j in range(_H // 16):
            zbuf[r, pl.ds(j * 16, 16)] = jnp.zeros((16,), jnp.float32)
        return carry
    lax.fori_loop(0, _ZR, _zrow, 0)

    nj = jnp.where(s < _NZCH % _NS, _NZCH // _NS + 1, _NZCH // _NS)

    def _zchunk(j, carry):
        off = (s + _NS * j) * _ZR
        pltpu.sync_copy(zbuf, acc_sh.at[pl.ds(off, _ZR)])
        return carry
    lax.fori_loop(0, nj, _zchunk, 0)
    plsc.subcore_barrier()

    pltpu.async_copy(h_hbm.at[src_v.at[0]], rows0, sem0)

    def _pair(i, carry):
        b = i * 2
        pltpu.async_copy(h_hbm.at[src_v.at[b + 1]], rows1, sem1)
        pltpu.make_async_copy(h_hbm.at[src_v.at[b]], rows0, sem0).wait()
        pltpu.sync_copy(rows0, acc_sh.at[dst_v.at[b]], add=True)
        pltpu.async_copy(h_hbm.at[src_v.at[b + 2]], rows0, sem0)
        pltpu.make_async_copy(h_hbm.at[src_v.at[b + 1]], rows1, sem1).wait()
        pltpu.sync_copy(rows1, acc_sh.at[dst_v.at[b + 1]], add=True)
        return carry
    lax.fori_loop(0, (_NCHUNK - 1) // 2, _pair, 0)

    last = _NCHUNK - 1
    pltpu.make_async_copy(h_hbm.at[src_v.at[last]], rows0, sem0).wait()
    pltpu.sync_copy(rows0, acc_sh.at[dst_v.at[last]], add=True)

    plsc.subcore_barrier()

    def _wchunk(j, carry):
        off = (s + _NS * j) * _ZR
        pltpu.sync_copy(acc_sh.at[pl.ds(off, _ZR)],
                        out_hbm.at[c].at[pl.ds(off, _ZR)])
        return carry
    lax.fori_loop(0, nj, _wchunk, 0)


@functools.cache
def _make_sc_agg_half(base):
    return pl.kernel(
        functools.partial(_sc_agg_half_body, base),
        out_type=jax.ShapeDtypeStruct((_NC, _NH, _H), jnp.float32),
        mesh=plsc.VectorSubcoreMesh(
            core_axis_name="c", subcore_axis_name="s",
            num_cores=_NC, num_subcores=_NS),
        scratch_types=[
            pltpu.VMEM((_NCHUNK, _K), jnp.int32),
            pltpu.VMEM((_NCHUNK, _K), jnp.int32),
            pltpu.VMEM((_NCHUNK, _K), jnp.int32),
            pltpu.VMEM((_K, _H), jnp.float32),
            pltpu.VMEM((_K, _H), jnp.float32),
            pltpu.VMEM((_ZR, _H), jnp.float32),
            pltpu.SemaphoreType.DMA,
            pltpu.SemaphoreType.DMA,
            pltpu.VMEM_SHARED((_NACC, _H), jnp.float32),
        ],
    )


def _sc_agg(pe, h):
    lo = _make_sc_agg_half(0)(pe, h)
    hi = _make_sc_agg_half(_NH)(pe, h)
    return lo, hi


def _emb_body(x_ref, emb_ref, o_ref):
    xv = x_ref[...]
    e0 = emb_ref[0:1, :]
    e1 = emb_ref[1:2, :]
    o_ref[...] = jnp.where(xv == 1, e1, e0)


def _emb_lookup(x2d, emb):
    return pl.pallas_call(
        _emb_body,
        out_shape=jax.ShapeDtypeStruct((_N, _H), jnp.float32),
    )(x2d, emb)


def _layer_body(h_ref, alo_ref, ahi_ref, w1_ref, w2_ref, g1_ref, b1_ref,
                g2_ref, b2_ref, o_ref):
    agg = jnp.concatenate([alo_ref[0] + alo_ref[1], ahi_ref[0] + ahi_ref[1]],
                          axis=0)
    z = h_ref[...] + agg
    z = jnp.dot(z, w1_ref[...], preferred_element_type=jnp.float32)
    mu = jnp.mean(z, axis=0, keepdims=True)
    var = jnp.mean((z - mu) * (z - mu), axis=0, keepdims=True)
    z = (z - mu) / jnp.sqrt(var + 1e-5) * g1_ref[...] + b1_ref[...]
    z = jnp.maximum(z, 0.0)
    z = jnp.dot(z, w2_ref[...], preferred_element_type=jnp.float32)
    mu = jnp.mean(z, axis=0, keepdims=True)
    var = jnp.mean((z - mu) * (z - mu), axis=0, keepdims=True)
    z = (z - mu) / jnp.sqrt(var + 1e-5) * g2_ref[...] + b2_ref[...]
    o_ref[...] = jnp.where(z > 0.0, z, jnp.exp(jnp.minimum(z, 0.0)) - 1.0)


def _layer(h, alo, ahi, w1, w2, g1, b1, g2, b2):
    return pl.pallas_call(
        _layer_body,
        out_shape=jax.ShapeDtypeStruct((_N, _H), jnp.float32),
    )(h, alo, ahi, w1, w2, g1, b1, g2, b2)


def _readout_body(h0_ref, h1_ref, h2_ref, h3_ref, wr1_ref, br1_ref, wr2_ref,
                  br2_ref, o_ref):
    acc = jnp.dot(h0_ref[...], wr1_ref[0], preferred_element_type=jnp.float32)
    acc += jnp.dot(h1_ref[...], wr1_ref[1], preferred_element_type=jnp.float32)
    acc += jnp.dot(h2_ref[...], wr1_ref[2], preferred_element_type=jnp.float32)
    acc += jnp.dot(h3_ref[...], wr1_ref[3], preferred_element_type=jnp.float32)
    acc = jnp.maximum(acc + br1_ref[...], 0.0)
    o_ref[...] = jnp.dot(acc, wr2_ref[...], preferred_element_type=jnp.float32) + br2_ref[...]


def _readout(h0, h1, h2, h3, wr1, br1, wr2, br2):
    blk = _N // 10
    row_spec = pl.BlockSpec((blk, _H), lambda i: (i, 0))
    return pl.pallas_call(
        _readout_body,
        grid=(10,),
        in_specs=[row_spec, row_spec, row_spec, row_spec,
                  pl.BlockSpec((_L + 1, _H, _H), lambda i: (0, 0, 0)),
                  pl.BlockSpec((1, _H), lambda i: (0, 0)),
                  pl.BlockSpec((_H, 1), lambda i: (0, 0)),
                  pl.BlockSpec((1, 1), lambda i: (0, 0))],
        out_specs=pl.BlockSpec((blk, 1), lambda i: (i, 0)),
        out_shape=jax.ShapeDtypeStruct((_N, 1), jnp.float32),
    )(h0, h1, h2, h3, wr1, br1, wr2, br2)


def kernel(x, edge_index, emb, W1, W2, bn1_g, bn1_b, bn2_g, bn2_b,
           Wr1, br1, Wr2, br2):
    x2d = x.astype(jnp.int32).reshape(_N, 1)
    src = edge_index[0].astype(jnp.int32)
    dst = edge_index[1].astype(jnp.int32)
    pe = (src | (dst << 16)).reshape(_NW, _NCHUNK, _K)

    h = _emb_lookup(x2d, emb)
    hidden = [h]
    for i in range(_L):
        alo, ahi = _sc_agg(pe, h)
        h = _layer(h, alo, ahi, W1[i], W2[i],
                   bn1_g[i:i + 1], bn1_b[i:i + 1],
                   bn2_g[i:i + 1], bn2_b[i:i + 1])
        hidden.append(h)

    wr1 = Wr1.reshape(_L + 1, _H, _H)
    return _readout(hidden[0], hidden[1], hidden[2], hidden[3],
                    wr1, br1.reshape(1, _H), Wr2, br2.reshape(1, 1))


# speedup vs baseline: 1.9315x; 1.9315x over previous
import functools

import jax
import jax.numpy as jnp
from jax import lax
from jax.experimental import pallas as pl
from jax.experimental.pallas import tpu as pltpu
from jax.experimental.pallas import tpu_sc as plsc

_N = 10000
_E = 320000
_H = 128
_L = 3

_NC = 2
_NS = 16
_NW = _NC * _NS
_EPT = _E // _NW
_K = 125
_NCHUNK = _EPT // _K
_ZR = 16
_NZCH = _N // _ZR


def _sc_agg_body(pe_hbm, h_hbm, out_hbm,
                 pi0, pi1, s0_v, d0_v, s1_v, d1_v, rows0, rows1, zbuf,
                 semi0, semi1, sem0, sem1, acc_sh):
    c = lax.axis_index("c")
    s = lax.axis_index("s")
    wid = s * _NC + c

    def _unp(pi, sv, dv):
        for g in range(-(-_K // 16)):
            lo = min(g * 16, _K - 16)
            sl = pl.ds(lo, 16)
            v = pi[sl]
            sv[sl] = lax.bitwise_and(v, 0xFFFF)
            dv[sl] = lax.shift_right_logical(v, 16)

    def _zrow(r, carry):
        for j in range(_H // 16):
            zbuf[r, pl.ds(j * 16, 16)] = jnp.zeros((16,), jnp.float32)
        return carry
    lax.fori_loop(0, _ZR, _zrow, 0)

    nj = jnp.where(s < _NZCH % _NS, _NZCH // _NS + 1, _NZCH // _NS)

    def _zchunk(j, carry):
        off = (s + _NS * j) * _ZR
        pltpu.sync_copy(zbuf, acc_sh.at[pl.ds(off, _ZR)])
        return carry
    lax.fori_loop(0, nj, _zchunk, 0)
    plsc.subcore_barrier()

    pltpu.sync_copy(pe_hbm.at[wid].at[0], pi0)
    _unp(pi0, s0_v, d0_v)
    pltpu.async_copy(h_hbm.at[s0_v], rows0, sem0)
    pltpu.async_copy(pe_hbm.at[wid].at[1], pi1, semi1)

    def _pair(i, carry):
        b = i * 2
        pltpu.make_async_copy(pe_hbm.at[wid].at[0], pi1, semi1).wait()
        _unp(pi1, s1_v, d1_v)
        pltpu.async_copy(h_hbm.at[s1_v], rows1, sem1)
        pltpu.async_copy(pe_hbm.at[wid].at[b + 2], pi0, semi0)
        pltpu.make_async_copy(h_hbm.at[s0_v], rows0, sem0).wait()
        pltpu.sync_copy(rows0, acc_sh.at[d0_v], add=True)
        pltpu.make_async_copy(pe_hbm.at[wid].at[0], pi0, semi0).wait()
        _unp(pi0, s0_v, d0_v)
        pltpu.async_copy(h_hbm.at[s0_v], rows0, sem0)
        pltpu.async_copy(pe_hbm.at[wid].at[b + 3], pi1, semi1)
        pltpu.make_async_copy(h_hbm.at[s1_v], rows1, sem1).wait()
        pltpu.sync_copy(rows1, acc_sh.at[d1_v], add=True)
        return carry
    lax.fori_loop(0, _NCHUNK // 2 - 1, _pair, 0)

    pltpu.make_async_copy(pe_hbm.at[wid].at[0], pi1, semi1).wait()
    _unp(pi1, s1_v, d1_v)
    pltpu.async_copy(h_hbm.at[s1_v], rows1, sem1)
    pltpu.make_async_copy(h_hbm.at[s0_v], rows0, sem0).wait()
    pltpu.sync_copy(rows0, acc_sh.at[d0_v], add=True)
    pltpu.make_async_copy(h_hbm.at[s1_v], rows1, sem1).wait()
    pltpu.sync_copy(rows1, acc_sh.at[d1_v], add=True)

    plsc.subcore_barrier()

    def _wchunk(j, carry):
        off = (s + _NS * j) * _ZR
        pltpu.sync_copy(acc_sh.at[pl.ds(off, _ZR)],
                        out_hbm.at[c].at[pl.ds(off, _ZR)])
        return carry
    lax.fori_loop(0, nj, _wchunk, 0)


@functools.cache
def _make_sc_agg():
    return pl.kernel(
        _sc_agg_body,
        out_type=jax.ShapeDtypeStruct((_NC, _N, _H), jnp.float32),
        mesh=plsc.VectorSubcoreMesh(
            core_axis_name="c", subcore_axis_name="s",
            num_cores=_NC, num_subcores=_NS),
        scratch_types=[
            pltpu.VMEM((_K,), jnp.int32),
            pltpu.VMEM((_K,), jnp.int32),
            pltpu.VMEM((_K,), jnp.int32),
            pltpu.VMEM((_K,), jnp.int32),
            pltpu.VMEM((_K,), jnp.int32),
            pltpu.VMEM((_K,), jnp.int32),
            pltpu.VMEM((_K, _H), jnp.float32),
            pltpu.VMEM((_K, _H), jnp.float32),
            pltpu.VMEM((_ZR, _H), jnp.float32),
            pltpu.SemaphoreType.DMA,
            pltpu.SemaphoreType.DMA,
            pltpu.SemaphoreType.DMA,
            pltpu.SemaphoreType.DMA,
            pltpu.VMEM_SHARED((_N, _H), jnp.float32),
        ],
    )


def _sc_agg(pe, h):
    return _make_sc_agg()(pe, h)


def _emb_body(x_ref, emb_ref, o_ref):
    xv = x_ref[...]
    e0 = emb_ref[0:1, :]
    e1 = emb_ref[1:2, :]
    o_ref[...] = jnp.where(xv == 1, e1, e0)


def _emb_lookup(x2d, emb):
    return pl.pallas_call(
        _emb_body,
        out_shape=jax.ShapeDtypeStruct((_N, _H), jnp.float32),
    )(x2d, emb)


def _layer_body(h_ref, a_ref, w1_ref, w2_ref, g1_ref, b1_ref, g2_ref, b2_ref,
                o_ref):
    z = h_ref[...] + a_ref[0] + a_ref[1]
    z = jnp.dot(z, w1_ref[...], preferred_element_type=jnp.float32)
    mu = jnp.mean(z, axis=0, keepdims=True)
    var = jnp.mean((z - mu) * (z - mu), axis=0, keepdims=True)
    z = (z - mu) / jnp.sqrt(var + 1e-5) * g1_ref[...] + b1_ref[...]
    z = jnp.maximum(z, 0.0)
    z = jnp.dot(z, w2_ref[...], preferred_element_type=jnp.float32)
    mu = jnp.mean(z, axis=0, keepdims=True)
    var = jnp.mean((z - mu) * (z - mu), axis=0, keepdims=True)
    z = (z - mu) / jnp.sqrt(var + 1e-5) * g2_ref[...] + b2_ref[...]
    o_ref[...] = jnp.where(z > 0.0, z, jnp.exp(jnp.minimum(z, 0.0)) - 1.0)


def _layer(h, agg, w1, w2, g1, b1, g2, b2):
    return pl.pallas_call(
        _layer_body,
        out_shape=jax.ShapeDtypeStruct((_N, _H), jnp.float32),
    )(h, agg, w1, w2, g1, b1, g2, b2)


def _readout_body(h0_ref, h1_ref, h2_ref, h3_ref, wr1_ref, br1_ref, wr2_ref,
                  br2_ref, o_ref):
    acc = jnp.dot(h0_ref[...], wr1_ref[0], preferred_element_type=jnp.float32)
    acc += jnp.dot(h1_ref[...], wr1_ref[1], preferred_element_type=jnp.float32)
    acc += jnp.dot(h2_ref[...], wr1_ref[2], preferred_element_type=jnp.float32)
    acc += jnp.dot(h3_ref[...], wr1_ref[3], preferred_element_type=jnp.float32)
    acc = jnp.maximum(acc + br1_ref[...], 0.0)
    o_ref[...] = jnp.dot(acc, wr2_ref[...], preferred_element_type=jnp.float32) + br2_ref[...]


def _readout(h0, h1, h2, h3, wr1, br1, wr2, br2):
    blk = _N // 10
    row_spec = pl.BlockSpec((blk, _H), lambda i: (i, 0))
    return pl.pallas_call(
        _readout_body,
        grid=(10,),
        in_specs=[row_spec, row_spec, row_spec, row_spec,
                  pl.BlockSpec((_L + 1, _H, _H), lambda i: (0, 0, 0)),
                  pl.BlockSpec((1, _H), lambda i: (0, 0)),
                  pl.BlockSpec((_H, 1), lambda i: (0, 0)),
                  pl.BlockSpec((1, 1), lambda i: (0, 0))],
        out_specs=pl.BlockSpec((blk, 1), lambda i: (i, 0)),
        out_shape=jax.ShapeDtypeStruct((_N, 1), jnp.float32),
    )(h0, h1, h2, h3, wr1, br1, wr2, br2)


def kernel(x, edge_index, emb, W1, W2, bn1_g, bn1_b, bn2_g, bn2_b,
           Wr1, br1, Wr2, br2):
    x2d = x.astype(jnp.int32).reshape(_N, 1)
    src = edge_index[0].astype(jnp.int32)
    dst = edge_index[1].astype(jnp.int32)
    pe = (src | (dst << 16)).reshape(_NW, _NCHUNK, _K)

    h = _emb_lookup(x2d, emb)
    hidden = [h]
    for i in range(_L):
        agg = _sc_agg(pe, h)
        h = _layer(h, agg, W1[i], W2[i],
                   bn1_g[i:i + 1], bn1_b[i:i + 1],
                   bn2_g[i:i + 1], bn2_b[i:i + 1])
        hidden.append(h)

    wr1 = Wr1.reshape(_L + 1, _H, _H)
    return _readout(hidden[0], hidden[1], hidden[2], hidden[3],
                    wr1, br1.reshape(1, _H), Wr2, br2.reshape(1, 1))
